# baseline (speedup 1.0000x reference)
import functools
import math

import jax
import jax.numpy as jnp
import numpy as np
from jax import lax
from jax.experimental import pallas as pl
from jax.experimental.pallas import tpu as pltpu
from jax.experimental.pallas import tpu_sc as plsc

_N = 1_000_000
_NS = 50_000
_STD = 0.05
_NW = 32
_BT = 1568
_NCH = 14
_CH = 112
_TOT = _NW * _BT
_NG = _BT // 16

_U32 = np.uint32


def _threefry2x32(k1, k2, x0, x1):
    rot = ((13, 15, 26, 6), (17, 29, 16, 24))

    def rotl(x, d):
        return ((x << _U32(d)) | (x >> _U32(32 - d))).astype(_U32)

    ks = (_U32(k1), _U32(k2), _U32(k1 ^ k2 ^ _U32(0x1BD11BDA)))
    x = [(x0 + ks[0]).astype(_U32), (x1 + ks[1]).astype(_U32)]

    def rounds(x, rs):
        for r in rs:
            a = (x[0] + x[1]).astype(_U32)
            b = (a ^ rotl(x[1], r)).astype(_U32)
            x = [a, b]
        return x

    for i, (ra, ka, kb) in enumerate(
        ((rot[0], 1, 2), (rot[1], 2, 0), (rot[0], 0, 1),
         (rot[1], 1, 2), (rot[0], 2, 0))
    ):
        x = rounds(x, ra)
        x = [(x[0] + ks[ka]).astype(_U32),
             (x[1] + ks[kb] + _U32(i + 1)).astype(_U32)]
    return x


def _random_bits32(key, n):
    counts_hi = np.zeros(n, dtype=_U32)
    counts_lo = np.arange(n, dtype=_U32)
    b1, b2 = _threefry2x32(key[0], key[1], counts_hi, counts_lo)
    return (b1 ^ b2).astype(_U32)


def _split_key(key):
    counts_hi = np.zeros(2, dtype=_U32)
    counts_lo = np.arange(2, dtype=_U32)
    b1, b2 = _threefry2x32(key[0], key[1], counts_hi, counts_lo)
    return (_U32(b1[0]), _U32(b2[0])), (_U32(b1[1]), _U32(b2[1]))


def _shuffle(key, n):
    x = np.arange(n, dtype=np.int32)
    num_rounds = int(np.ceil(3 * np.log(max(1, n)) / np.log(2**32 - 1)))
    for _ in range(num_rounds):
        key, subkey = _split_key(key)
        sort_keys = _random_bits32(subkey, n)
        x = x[np.argsort(sort_keys, kind="stable")]
    return x


def _ndtri(p):
    a = (-3.969683028665376e+01, 2.209460984245205e+02, -2.759285104469687e+02,
         1.383577518672690e+02, -3.066479806614716e+01, 2.506628277459239e+00)
    b = (-5.447609879822406e+01, 1.615858368580409e+02, -1.556989798598866e+02,
         6.680131188771972e+01, -1.328068155288572e+01)
    c = (-7.784894002430293e-03, -3.223964580411365e-01, -2.400758277161838e+00,
         -2.549732539343734e+00, 4.374664141464968e+00, 2.938163982698783e+00)
    d = (7.784695709041462e-03, 3.224671290700398e-01, 2.445134137142996e+00,
         3.754408661907416e+00)
    p = np.asarray(p, dtype=np.float64)
    out = np.empty_like(p)
    plow, phigh = 0.02425, 1 - 0.02425
    lo = p < plow
    hi = p > phigh
    mid = ~(lo | hi)
    q = np.sqrt(-2 * np.log(np.where(lo, p, 0.5)))
    out_lo = ((((((c[0] * q + c[1]) * q + c[2]) * q + c[3]) * q + c[4]) * q + c[5])
              / ((((d[0] * q + d[1]) * q + d[2]) * q + d[3]) * q + 1))
    q = p - 0.5
    r = q * q
    out_mid = ((((((a[0] * r + a[1]) * r + a[2]) * r + a[3]) * r + a[4]) * r + a[5]) * q
               / (((((b[0] * r + b[1]) * r + b[2]) * r + b[3]) * r + b[4]) * r + 1))
    q = np.sqrt(-2 * np.log(np.where(hi, 1 - p, 0.5)))
    out_hi = -((((((c[0] * q + c[1]) * q + c[2]) * q + c[3]) * q + c[4]) * q + c[5])
               / ((((d[0] * q + d[1]) * q + d[2]) * q + d[3]) * q + 1))
    out[lo] = out_lo[lo]
    out[mid] = out_mid[mid]
    out[hi] = out_hi[hi]
    return out


def _normal_f32(key, shape):
    n = int(np.prod(shape))
    bits = _random_bits32(key, n)
    float_bits = ((bits >> _U32(9)) | _U32(0x3F800000)).astype(_U32)
    u01 = float_bits.view(np.float32) - np.float32(1.0)
    lo = np.float32(np.nextafter(np.float32(-1.0), np.float32(0.0)))
    hi = np.float32(1.0)
    u = np.maximum(lo, (u01 * (hi - lo) + lo).astype(np.float32))
    erfinv = _ndtri((u.astype(np.float64) + 1.0) * 0.5) / math.sqrt(2.0)
    return (np.float32(math.sqrt(2.0)) * erfinv.astype(np.float32)).reshape(shape)


def _build_consts():
    seed_key = (_U32(0), _U32(42))
    k_perm, k_noise = _split_key(seed_key)
    sel = _shuffle(k_perm, _N)[:_NS].astype(np.int32)
    noise = _normal_f32(k_noise, (_NS, 3)) * np.float32(_STD)
    idx_pad = np.zeros((_TOT,), dtype=np.int32)
    idx_pad[:_NS] = sel
    idx_w = np.ascontiguousarray(idx_pad.reshape(_NW, _NCH, _CH))
    noise_cm = np.zeros((3, _TOT), dtype=np.float32)
    noise_cm[:, :_NS] = noise.T
    noise_w = np.ascontiguousarray(
        noise_cm.reshape(3, _NW, _BT).transpose(1, 0, 2)).reshape(_NW, 1, 3 * _BT)
    return sel, idx_w, noise_w


_SEL, _IDX2D, _NOISE_CM = _build_consts()


@functools.cache
def _make_sc_sample():
    return functools.partial(
        pl.kernel,
        mesh=plsc.VectorSubcoreMesh(core_axis_name="c", subcore_axis_name="s"),
        out_type=jax.ShapeDtypeStruct((_NW, 1, 4 * _BT), jnp.float32),
        compiler_params=pltpu.CompilerParams(
            use_tc_tiling_on_sc=False, needs_layout_passes=False),
        scratch_types=[
            pltpu.VMEM((_NCH, _CH), jnp.int32),
            pltpu.VMEM((_BT, 3), jnp.float32),
            pltpu.VMEM((3 * _BT,), jnp.float32),
            pltpu.VMEM((4 * _BT,), jnp.float32),
            pltpu.VMEM((16,), jnp.float32),
            pltpu.SemaphoreType.DMA,
        ],
    )(_sc_sample)


def _sc_sample(xyz_hbm, t16_hbm, idx_hbm, noise_hbm, out_hbm,
               idx_v, rows_v, noise_v, out_v, t_v, sem):
    wid = lax.axis_index("s") * 2 + lax.axis_index("c")

    pltpu.sync_copy(idx_hbm.at[wid], idx_v)
    pltpu.sync_copy(noise_hbm.at[wid, 0], noise_v)
    pltpu.sync_copy(t16_hbm, t_v)

    copies = [
        pltpu.async_copy(
            xyz_hbm.at[idx_v.at[j]],
            rows_v.at[pl.ds(j * _CH, _CH)],
            sem,
        )
        for j in range(_NCH)
    ]
    for cp in copies:
        cp.wait()

    lane = lax.iota(jnp.int32, 16)
    tvec = t_v[...]
    cols = [jnp.full((16,), c, jnp.int32) for c in range(3)]

    def body(g, carry):
        off = pl.multiple_of(g * 16, 8)
        r16 = g * 16 + lane
        obase = r16 * 4
        for c in range(3):
            v = plsc.load_gather(rows_v, [r16, cols[c]])
            n = noise_v[pl.ds(c * _BT + off, 16)]
            res = jnp.clip(v + n, -10.0, 10.0)
            plsc.store_scatter(out_v, [obase + c], res)
        plsc.store_scatter(out_v, [obase + 3], tvec)
        return carry

    lax.fori_loop(0, _NG, body, 0)

    pltpu.sync_copy(out_v, out_hbm.at[wid, 0])


def kernel(xyz, t, iteration):
    t16 = jnp.broadcast_to(t.astype(jnp.float32).reshape(1), (16,))
    out = _make_sc_sample()(xyz, t16, jnp.asarray(_IDX2D), jnp.asarray(_NOISE_CM))
    collocation_points = jnp.reshape(out, (_TOT, 4))[:_NS]
    return (collocation_points, jnp.asarray(_SEL))


# speedup vs baseline: 23.4026x; 23.4026x over previous
import functools
import math

import jax
import jax.numpy as jnp
import numpy as np
from jax import lax
from jax.experimental import pallas as pl
from jax.experimental.pallas import tpu as pltpu
from jax.experimental.pallas import tpu_sc as plsc

_N = 1_000_000
_NS = 50_000
_STD = 0.05
_NW = 32
_BT = 1568
_NCH = 14
_CH = 112
_TOT = _NW * _BT
_NG = _BT // 16

_U32 = np.uint32


def _threefry2x32(k1, k2, x0, x1):
    rot = ((13, 15, 26, 6), (17, 29, 16, 24))

    def rotl(x, d):
        return ((x << _U32(d)) | (x >> _U32(32 - d))).astype(_U32)

    ks = (_U32(k1), _U32(k2), _U32(k1 ^ k2 ^ _U32(0x1BD11BDA)))
    x = [(x0 + ks[0]).astype(_U32), (x1 + ks[1]).astype(_U32)]

    def rounds(x, rs):
        for r in rs:
            a = (x[0] + x[1]).astype(_U32)
            b = (a ^ rotl(x[1], r)).astype(_U32)
            x = [a, b]
        return x

    for i, (ra, ka, kb) in enumerate(
        ((rot[0], 1, 2), (rot[1], 2, 0), (rot[0], 0, 1),
         (rot[1], 1, 2), (rot[0], 2, 0))
    ):
        x = rounds(x, ra)
        x = [(x[0] + ks[ka]).astype(_U32),
             (x[1] + ks[kb] + _U32(i + 1)).astype(_U32)]
    return x


def _random_bits32(key, n):
    counts_hi = np.zeros(n, dtype=_U32)
    counts_lo = np.arange(n, dtype=_U32)
    b1, b2 = _threefry2x32(key[0], key[1], counts_hi, counts_lo)
    return (b1 ^ b2).astype(_U32)


def _split_key(key):
    counts_hi = np.zeros(2, dtype=_U32)
    counts_lo = np.arange(2, dtype=_U32)
    b1, b2 = _threefry2x32(key[0], key[1], counts_hi, counts_lo)
    return (_U32(b1[0]), _U32(b2[0])), (_U32(b1[1]), _U32(b2[1]))


def _shuffle(key, n):
    x = np.arange(n, dtype=np.int32)
    num_rounds = int(np.ceil(3 * np.log(max(1, n)) / np.log(2**32 - 1)))
    for _ in range(num_rounds):
        key, subkey = _split_key(key)
        sort_keys = _random_bits32(subkey, n)
        x = x[np.argsort(sort_keys, kind="stable")]
    return x


def _ndtri(p):
    a = (-3.969683028665376e+01, 2.209460984245205e+02, -2.759285104469687e+02,
         1.383577518672690e+02, -3.066479806614716e+01, 2.506628277459239e+00)
    b = (-5.447609879822406e+01, 1.615858368580409e+02, -1.556989798598866e+02,
         6.680131188771972e+01, -1.328068155288572e+01)
    c = (-7.784894002430293e-03, -3.223964580411365e-01, -2.400758277161838e+00,
         -2.549732539343734e+00, 4.374664141464968e+00, 2.938163982698783e+00)
    d = (7.784695709041462e-03, 3.224671290700398e-01, 2.445134137142996e+00,
         3.754408661907416e+00)
    p = np.asarray(p, dtype=np.float64)
    out = np.empty_like(p)
    plow, phigh = 0.02425, 1 - 0.02425
    lo = p < plow
    hi = p > phigh
    mid = ~(lo | hi)
    q = np.sqrt(-2 * np.log(np.where(lo, p, 0.5)))
    out_lo = ((((((c[0] * q + c[1]) * q + c[2]) * q + c[3]) * q + c[4]) * q + c[5])
              / ((((d[0] * q + d[1]) * q + d[2]) * q + d[3]) * q + 1))
    q = p - 0.5
    r = q * q
    out_mid = ((((((a[0] * r + a[1]) * r + a[2]) * r + a[3]) * r + a[4]) * r + a[5]) * q
               / (((((b[0] * r + b[1]) * r + b[2]) * r + b[3]) * r + b[4]) * r + 1))
    q = np.sqrt(-2 * np.log(np.where(hi, 1 - p, 0.5)))
    out_hi = -((((((c[0] * q + c[1]) * q + c[2]) * q + c[3]) * q + c[4]) * q + c[5])
               / ((((d[0] * q + d[1]) * q + d[2]) * q + d[3]) * q + 1))
    out[lo] = out_lo[lo]
    out[mid] = out_mid[mid]
    out[hi] = out_hi[hi]
    return out


def _normal_f32(key, shape):
    n = int(np.prod(shape))
    bits = _random_bits32(key, n)
    float_bits = ((bits >> _U32(9)) | _U32(0x3F800000)).astype(_U32)
    u01 = float_bits.view(np.float32) - np.float32(1.0)
    lo = np.float32(np.nextafter(np.float32(-1.0), np.float32(0.0)))
    hi = np.float32(1.0)
    u = np.maximum(lo, (u01 * (hi - lo) + lo).astype(np.float32))
    erfinv = _ndtri((u.astype(np.float64) + 1.0) * 0.5) / math.sqrt(2.0)
    return (np.float32(math.sqrt(2.0)) * erfinv.astype(np.float32)).reshape(shape)


def _build_consts():
    seed_key = (_U32(0), _U32(42))
    k_perm, k_noise = _split_key(seed_key)
    sel = _shuffle(k_perm, _N)[:_NS].astype(np.int32)
    noise = _normal_f32(k_noise, (_NS, 3)) * np.float32(_STD)
    idx_pad = np.zeros((_TOT,), dtype=np.int32)
    idx_pad[:_NS] = sel
    idx_w = np.ascontiguousarray(idx_pad.reshape(_NW, _NCH, _CH))
    noise_cm = np.zeros((3, _TOT), dtype=np.float32)
    noise_cm[:, :_NS] = noise.T
    noise_w = np.ascontiguousarray(
        noise_cm.reshape(3, _NW, _BT).transpose(1, 0, 2)).reshape(_NW, 1, 3 * _BT)
    return sel, idx_w, noise_w


_SEL, _IDX2D, _NOISE_CM = _build_consts()


@functools.cache
def _make_sc_sample():
    return functools.partial(
        pl.kernel,
        mesh=plsc.VectorSubcoreMesh(core_axis_name="c", subcore_axis_name="s"),
        out_type=jax.ShapeDtypeStruct((_NW, 1, 4 * _BT), jnp.float32),
        compiler_params=pltpu.CompilerParams(
            use_tc_tiling_on_sc=False, needs_layout_passes=False),
        scratch_types=[
            pltpu.VMEM((_NCH, _CH), jnp.int32),
            pltpu.VMEM((3 * _BT,), jnp.float32),
            pltpu.VMEM((3 * _BT,), jnp.float32),
            pltpu.VMEM((4 * _BT,), jnp.float32),
            pltpu.VMEM((16,), jnp.float32),
            pltpu.SemaphoreType.DMA,
        ],
    )(_sc_sample)


def _sc_sample(x_hbm, y_hbm, z_hbm, t16_hbm, idx_hbm, noise_hbm, out_hbm,
               idx_v, comp_v, noise_v, out_v, t_v, sem):
    wid = lax.axis_index("s") * 2 + lax.axis_index("c")

    pltpu.sync_copy(idx_hbm.at[wid], idx_v)
    pltpu.sync_copy(noise_hbm.at[wid, 0], noise_v)
    pltpu.sync_copy(t16_hbm, t_v)

    copies = [
        pltpu.async_copy(
            tab.at[idx_v.at[j]],
            comp_v.at[pl.ds(c * _BT + j * _CH, _CH)],
            sem,
        )
        for c, tab in enumerate((x_hbm, y_hbm, z_hbm))
        for j in range(_NCH)
    ]
    for cp in copies:
        cp.wait()

    lane = lax.iota(jnp.int32, 16)
    tvec = t_v[...]

    def body(g, carry):
        off = pl.multiple_of(g * 16, 8)
        obase = (g * 16 + lane) * 4
        for c in range(3):
            v = comp_v[pl.ds(c * _BT + off, 16)]
            n = noise_v[pl.ds(c * _BT + off, 16)]
            res = jnp.clip(v + n, -10.0, 10.0)
            plsc.store_scatter(out_v, [obase + c], res)
        plsc.store_scatter(out_v, [obase + 3], tvec)
        return carry

    lax.fori_loop(0, _NG, body, 0)

    pltpu.sync_copy(out_v, out_hbm.at[wid, 0])


def kernel(xyz, t, iteration):
    t16 = jnp.broadcast_to(t.astype(jnp.float32).reshape(1), (16,))
    xc, yc, zc = xyz[:, 0], xyz[:, 1], xyz[:, 2]
    out = _make_sc_sample()(xc, yc, zc, t16,
                            jnp.asarray(_IDX2D), jnp.asarray(_NOISE_CM))
    collocation_points = jnp.reshape(out, (_TOT, 4))[:_NS]
    return (collocation_points, jnp.asarray(_SEL))
